# baseline (speedup 1.0000x reference)
import functools

import jax
import jax.numpy as jnp
from jax import lax
from jax.experimental import pallas as pl
from jax.experimental.pallas import tpu as pltpu
from jax.experimental.pallas import tpu_sc as plsc

N = 10000
E = 160000
D = 256
NC = 2
NS = 16
NW = NC * NS
NPW = 320
NBP = NW * NPW
ECH = E // NW
ECH_IT = ECH // 16
ECH_TAIL = ECH - ECH_IT * 16
GB = 128
ECAP = E + NW * GB
BLK = 4000
NBLK = E // BLK
SCAN_IT = BLK // 16
PEND = BLK + 2 * GB
CH = 1024
HG = GB // 2
MMB_A = 1000
MMB_B = 1024

_mesh = plsc.VectorSubcoreMesh(core_axis_name="c", subcore_axis_name="s")
_sc_params = pltpu.CompilerParams(needs_layout_passes=False)


def _wid():
    return lax.axis_index("s") * NC + lax.axis_index("c")


def _deg_body(s0, d0, s1, d1, s2, d2, hist_out, hist, ebuf):
    w = _wid()
    zero16 = jnp.zeros((16,), jnp.float32)

    def zloop(i):
        hist[pl.ds(i * 16, 16)] = zero16

    plsc.parallel_loop(0, (6 * NBP) // 16, 1, unroll=8)(zloop)
    ones = jnp.ones((16,), jnp.float32)
    full_m = jnp.ones((16,), jnp.bool_)
    tail_m = lax.iota(jnp.int32, 16) < ECH_TAIL
    arrs = [s0, d0, s1, d1, s2, d2]
    for a in range(6):
        base = a * NBP
        pltpu.sync_copy(arrs[a].at[pl.ds(w * ECH, ECH)], ebuf.at[pl.ds(0, ECH)])

        def body(i, base=base):
            idx = ebuf[pl.ds(i * 16, 16)] + base
            plsc.addupdate_scatter(hist, [idx], ones, mask=full_m)

        plsc.parallel_loop(0, ECH_IT, 1, unroll=4)(body)
        idx = ebuf[pl.ds(ECH_IT * 16, 16)] + base
        plsc.addupdate_scatter(hist, [idx], ones, mask=tail_m)
    pltpu.sync_copy(hist, hist_out.at[pl.ds(w * 6 * NBP, 6 * NBP)])


_deg = pl.kernel(
    _deg_body,
    out_type=jax.ShapeDtypeStruct((NW * 6 * NBP,), jnp.float32),
    mesh=_mesh,
    compiler_params=_sc_params,
    scratch_types=[
        pltpu.VMEM((6 * NBP,), jnp.float32),
        pltpu.VMEM((ECH + 16,), jnp.int32),
    ],
)


def _norm_body(part, norms, counts, buf, nbuf, cbuf, dsem):
    w = _wid()
    off = w * NPW
    for a in range(6):
        cps = [pltpu.async_copy(
                   part.at[pl.ds(k * 6 * NBP + a * NBP + off, NPW)],
                   buf.at[pl.ds(k * NPW, NPW)], dsem) for k in range(NW)]
        for cp in cps:
            cp.wait()
        vacc = jnp.zeros((16,), jnp.float32)
        for i in range(NPW // 16):
            s = buf[pl.ds(i * 16, 16)]
            for k in range(1, NW):
                s = s + buf[pl.ds(k * NPW + i * 16, 16)]
            vacc = vacc + s
            x = jnp.maximum(s, 1.0)
            ii = plsc.bitcast(x, jnp.int32)
            yi = 0x5F3759DF - lax.shift_right_logical(ii, 1)
            y = plsc.bitcast(yi, jnp.float32)
            for _ in range(3):
                y = y * (1.5 - 0.5 * x * y * y)
            nbuf[pl.ds(i * 16, 16)] = y
        pltpu.sync_copy(nbuf, norms.at[pl.ds(a * NBP + off, NPW)])
        if a % 2 == 1:
            r = a // 2
            cbuf[pl.ds(0, 16)] = vacc
            pltpu.sync_copy(cbuf, counts.at[pl.ds((r * NW + w) * 16, 16)])


_norm = pl.kernel(
    _norm_body,
    out_type=(jax.ShapeDtypeStruct((6 * NBP,), jnp.float32),
              jax.ShapeDtypeStruct((3 * NW * 16,), jnp.float32)),
    mesh=_mesh,
    compiler_params=_sc_params,
    scratch_types=[
        pltpu.VMEM((NW * NPW,), jnp.float32),
        pltpu.VMEM((NPW,), jnp.float32),
        pltpu.VMEM((16,), jnp.float32),
        pltpu.SemaphoreType.DMA,
    ],
)


def _range_offsets(cbuf, r, w):
    goff = jnp.int32(0)
    ng_w = jnp.int32(0)
    for k in range(NW):
        ck = jnp.sum(cbuf[pl.ds((r * NW + k) * 16, 16)]).astype(jnp.int32)
        ng_k = lax.shift_right_logical(ck + (GB - 1), 7)
        goff = goff + jnp.where(k < w, ng_k, 0)
        ng_w = ng_w + jnp.where(k == w, ng_k, 0)
    return goff, ng_w


def _part_body(norms, counts, s0, d0, s1, d1, s2, d2, w0, w1, w2,
               srcp, dstlp, wfp,
               nflat, cbuf, sblk, dblk, eblk, ps, pd, pw, sem):
    w = _wid()
    lo = w * NPW
    zero16 = jnp.zeros((16,), jnp.float32)
    zi16 = jnp.zeros((16,), jnp.int32)
    pltpu.sync_copy(counts, cbuf)
    srcs = [s0, s1, s2]
    dsts = [d0, d1, d2]
    ws = [w0, w1, w2]
    for r in range(3):
        pltpu.sync_copy(norms.at[pl.ds(2 * r * NBP, NBP)], nflat.at[pl.ds(0, NBP)])
        pltpu.sync_copy(norms.at[pl.ds((2 * r + 1) * NBP, NBP)],
                        nflat.at[pl.ds(NBP, NBP)])
        goff, _ = _range_offsets(cbuf, r, w)
        ebase = r * ECAP + goff * GB

        def blk_body(b, carry, r=r, ebase=ebase):
            rem, flushed = carry
            e0 = b * BLK
            c1 = pltpu.async_copy(srcs[r].at[pl.ds(e0, BLK)], sblk, sem)
            c2 = pltpu.async_copy(dsts[r].at[pl.ds(e0, BLK)], dblk, sem)
            c3 = pltpu.async_copy(ws[r].at[pl.ds(e0, BLK)], eblk, sem)
            c1.wait()
            c2.wait()
            c3.wait()

            def scan(i, base):
                d = dblk[pl.ds(i * 16, 16)]
                m = (d >= lo) & (d < lo + NPW)
                mi = m.astype(jnp.int32)
                pos = base + plsc.cumsum(mi) - mi
                s = sblk[pl.ds(i * 16, 16)]
                ew = eblk[pl.ds(i * 16, 16)]
                ns = plsc.load_gather(nflat, [s])
                nd = plsc.load_gather(nflat, [d + NBP])
                plsc.store_scatter(ps, [pos], s, mask=m)
                plsc.store_scatter(pd, [pos], d - lo, mask=m)
                plsc.store_scatter(pw, [pos], ew * ns * nd, mask=m)
                return base + plsc.all_reduce_population_count(m)

            base = plsc.parallel_loop(
                0, SCAN_IT, 1, unroll=4,
                carry=jnp.zeros((16,), jnp.int32) + rem)(scan)
            cnt = jnp.max(base)
            ngf = lax.shift_right_logical(cnt, 7)

            def flush(g, _, ebase=ebase):
                eout = ebase + (flushed + g) * GB
                c1 = pltpu.async_copy(ps.at[pl.ds(g * GB, GB)],
                                      srcp.at[pl.ds(eout, GB)], sem)
                c2 = pltpu.async_copy(pd.at[pl.ds(g * GB, GB)],
                                      dstlp.at[pl.ds(eout, GB)], sem)
                c3 = pltpu.async_copy(pw.at[pl.ds(g * GB, GB)],
                                      wfp.at[pl.ds(eout, GB)], sem)
                c1.wait()
                c2.wait()
                c3.wait()
                return 0

            lax.fori_loop(0, ngf, flush, 0)
            tail0 = ngf * GB
            for i in range(GB // 16):
                ps[pl.ds(i * 16, 16)] = ps[pl.ds(tail0 + i * 16, 16)]
                pd[pl.ds(i * 16, 16)] = pd[pl.ds(tail0 + i * 16, 16)]
                pw[pl.ds(i * 16, 16)] = pw[pl.ds(tail0 + i * 16, 16)]
            return (cnt & (GB - 1), flushed + ngf)

        rem, flushed = lax.fori_loop(0, NBLK, blk_body,
                                     (jnp.int32(0), jnp.int32(0)))
        for i in range(GB // 16):
            ps[pl.ds(rem + i * 16, 16)] = zi16
            pd[pl.ds(rem + i * 16, 16)] = zi16
            pw[pl.ds(rem + i * 16, 16)] = zero16

        @pl.when(rem > 0)
        def _flush_tail(ebase=ebase, rem=rem, flushed=flushed):
            eout = ebase + flushed * GB
            c1 = pltpu.async_copy(ps.at[pl.ds(0, GB)],
                                  srcp.at[pl.ds(eout, GB)], sem)
            c2 = pltpu.async_copy(pd.at[pl.ds(0, GB)],
                                  dstlp.at[pl.ds(eout, GB)], sem)
            c3 = pltpu.async_copy(pw.at[pl.ds(0, GB)],
                                  wfp.at[pl.ds(eout, GB)], sem)
            c1.wait()
            c2.wait()
            c3.wait()


_part = pl.kernel(
    _part_body,
    out_type=(jax.ShapeDtypeStruct((3 * ECAP,), jnp.int32),
              jax.ShapeDtypeStruct((3 * ECAP,), jnp.int32),
              jax.ShapeDtypeStruct((3 * ECAP,), jnp.float32)),
    mesh=_mesh,
    compiler_params=_sc_params,
    scratch_types=[
        pltpu.VMEM((2 * NBP,), jnp.float32),
        pltpu.VMEM((3 * NW * 16,), jnp.float32),
        pltpu.VMEM((BLK,), jnp.int32),
        pltpu.VMEM((BLK,), jnp.int32),
        pltpu.VMEM((BLK,), jnp.float32),
        pltpu.VMEM((PEND,), jnp.int32),
        pltpu.VMEM((PEND,), jnp.int32),
        pltpu.VMEM((PEND,), jnp.float32),
        pltpu.SemaphoreType.DMA,
    ],
)


def _agg_body(do_relu, Z0, Z1, Z2, srcp, dstlp, wfp, counts, b0, b1, b2,
              out, acc, cbuf, sl, dl, wl, stage, bbuf, sem, sem_a, sem_b):
    w = _wid()
    lo = w * NPW
    zero16 = jnp.zeros((16,), jnp.float32)
    iota16 = lax.iota(jnp.int32, 16)

    def za(i):
        acc[pl.ds(i * 16, 16)] = zero16

    plsc.parallel_loop(0, (NPW * D) // 16, 1, unroll=8)(za)
    pltpu.sync_copy(counts, cbuf)
    Zs = [Z0, Z1, Z2]
    for r in range(3):
        goff, ng_w = _range_offsets(cbuf, r, w)
        ebase = r * ECAP + goff * GB
        nch = lax.shift_right_logical(ng_w * GB + (CH - 1), 10)

        def ch_body(c, _, r=r, ebase=ebase, ng_w=ng_w):
            c0 = ebase + c * CH
            d1_ = pltpu.async_copy(srcp.at[pl.ds(c0, CH)], sl, sem)
            d2_ = pltpu.async_copy(dstlp.at[pl.ds(c0, CH)], dl, sem)
            d3_ = pltpu.async_copy(wfp.at[pl.ds(c0, CH)], wl, sem)
            d1_.wait()
            d2_.wait()
            d3_.wait()
            ngc = jnp.minimum(ng_w - c * (CH // GB), CH // GB)
            nh = ngc * 2

            def fire(h, slot, r=r):
                sm = sem_a if slot == 0 else sem_b
                pltpu.async_copy(
                    Zs[r].at[sl.at[pl.ds(h * HG, HG)]],
                    stage.at[pl.ds(slot * HG, HG), :], sm)

            def drain(slot, r=r):
                sm = sem_a if slot == 0 else sem_b
                pltpu.make_async_copy(
                    Zs[r].at[pl.ds(0, HG), :],
                    stage.at[pl.ds(slot * HG, HG), :], sm).wait()

            def accum(h, slot):
                def ed(j):
                    js = jnp.full((16,), h * HG + j, jnp.int32)
                    wb = plsc.load_gather(wl, [js])
                    db = plsc.load_gather(dl, [js]) * D
                    rowv = jnp.full((16,), slot * HG + j, jnp.int32)
                    colv = iota16
                    for k in range(16):
                        v = plsc.load_gather(stage, [rowv, colv])
                        plsc.addupdate_scatter(acc, [db + colv], v * wb)
                        colv = colv + 16

                plsc.parallel_loop(0, HG, 1, unroll=6)(ed)

            fire(0, 0)

            def pair(g, _):
                h0 = g * 2
                drain(0)
                fire(h0 + 1, 1)
                accum(h0, 0)
                drain(1)

                @pl.when(g + 1 < ngc)
                def _():
                    fire(h0 + 2, 0)

                accum(h0 + 1, 1)
                return 0

            lax.fori_loop(0, ngc, pair, 0)
            return 0

        lax.fori_loop(0, nch, ch_body, 0)

    pltpu.sync_copy(b0, bbuf.at[pl.ds(0, D)])
    pltpu.sync_copy(b1, bbuf.at[pl.ds(D, D)])
    pltpu.sync_copy(b2, bbuf.at[pl.ds(2 * D, D)])
    for k in range(16):
        bbuf[pl.ds(3 * D + k * 16, 16)] = (
            bbuf[pl.ds(k * 16, 16)] + bbuf[pl.ds(D + k * 16, 16)]
            + bbuf[pl.ds(2 * D + k * 16, 16)])

    def ep(t, _):
        rbase = t * D
        for k in range(16):
            v = acc[pl.ds(rbase + k * 16, 16)] + bbuf[pl.ds(3 * D + k * 16, 16)]
            if do_relu:
                v = jnp.maximum(v, 0.0)
            acc[pl.ds(rbase + k * 16, 16)] = v
        return 0

    lax.fori_loop(0, NPW, ep, 0)
    pltpu.sync_copy(acc, out.at[pl.ds(lo * D, NPW * D)])


def _make_agg(do_relu):
    return pl.kernel(
        functools.partial(_agg_body, do_relu),
        out_type=jax.ShapeDtypeStruct((NBP * D,), jnp.float32),
        mesh=_mesh,
        compiler_params=_sc_params,
        scratch_types=[
            pltpu.VMEM((NPW * D,), jnp.float32),
            pltpu.VMEM((3 * NW * 16,), jnp.float32),
            pltpu.VMEM((CH,), jnp.int32),
            pltpu.VMEM((CH,), jnp.int32),
            pltpu.VMEM((CH,), jnp.float32),
            pltpu.VMEM((GB, D), jnp.float32),
            pltpu.VMEM((4 * D,), jnp.float32),
            pltpu.SemaphoreType.DMA,
            pltpu.SemaphoreType.DMA,
            pltpu.SemaphoreType.DMA,
        ],
    )


_agg_relu = _make_agg(True)
_agg_plain = _make_agg(False)


def _mm_body(x_ref, w0_ref, w1_ref, w2_ref, o0_ref, o1_ref, o2_ref):
    xb = x_ref[...]
    o0_ref[...] = jnp.dot(xb, w0_ref[...], preferred_element_type=jnp.float32)
    o1_ref[...] = jnp.dot(xb, w1_ref[...], preferred_element_type=jnp.float32)
    o2_ref[...] = jnp.dot(xb, w2_ref[...], preferred_element_type=jnp.float32)


def _make_mm(m, bm):
    wspec = pl.BlockSpec((D, D), lambda i: (0, 0))
    ospec = pl.BlockSpec((bm, D), lambda i: (i, 0))
    return pl.pallas_call(
        _mm_body,
        grid=(m // bm,),
        in_specs=[pl.BlockSpec((bm, D), lambda i: (i, 0)), wspec, wspec, wspec],
        out_specs=[ospec, ospec, ospec],
        out_shape=[jax.ShapeDtypeStruct((m, D), jnp.float32)] * 3,
    )


_mm_a = _make_mm(N, MMB_A)
_mm_b = _make_mm(NBP, MMB_B)


def kernel(x, edge_index_r0, edge_index_r1, edge_index_r2, w_r0, w_r1, w_r2,
           W1_r0, b1_r0, W1_r1, b1_r1, W1_r2, b1_r2,
           W2_r0, b2_r0, W2_r1, b2_r1, W2_r2, b2_r2):
    sd = (edge_index_r0[0], edge_index_r0[1], edge_index_r1[0],
          edge_index_r1[1], edge_index_r2[0], edge_index_r2[1])
    part = _deg(*sd)
    norms, counts = _norm(part)
    srcp, dstlp, wfp = _part(norms, counts, *sd, w_r0, w_r1, w_r2)
    z0, z1, z2 = _mm_a(x, W1_r0, W1_r1, W1_r2)
    hflat = _agg_relu(z0, z1, z2, srcp, dstlp, wfp, counts,
                      b1_r0, b1_r1, b1_r2)
    h = hflat.reshape(NBP, D)
    y0, y1, y2 = _mm_b(h, W2_r0, W2_r1, W2_r2)
    oflat = _agg_plain(y0, y1, y2, srcp, dstlp, wfp, counts,
                       b2_r0, b2_r1, b2_r2)
    return oflat.reshape(NBP, D)[:N]


# speedup vs baseline: 1.1707x; 1.1707x over previous
import functools

import jax
import jax.numpy as jnp
from jax import lax
from jax.experimental import pallas as pl
from jax.experimental.pallas import tpu as pltpu
from jax.experimental.pallas import tpu_sc as plsc

N = 10000
E = 160000
D = 256
NC = 2
NS = 16
NW = NC * NS
NPW = 320
NBP = NW * NPW
ECH = E // NW
ECH_IT = ECH // 16
ECH_TAIL = ECH - ECH_IT * 16
GB = 128
ECAP = E + NW * GB
BLK = 4000
NBLK = E // BLK
SCAN_IT = BLK // 16
PEND = BLK + 2 * GB
CH = 1024
HG = GB // 2
MMB_A = 1000
MMB_B = 1024

_mesh = plsc.VectorSubcoreMesh(core_axis_name="c", subcore_axis_name="s")
_sc_params = pltpu.CompilerParams(needs_layout_passes=False)


def _wid():
    return lax.axis_index("s") * NC + lax.axis_index("c")


def _deg_body(s0, d0, s1, d1, s2, d2, hist_out, hist, ebuf):
    w = _wid()
    zero16 = jnp.zeros((16,), jnp.float32)

    def zloop(i):
        hist[pl.ds(i * 16, 16)] = zero16

    plsc.parallel_loop(0, (6 * NBP) // 16, 1, unroll=8)(zloop)
    ones = jnp.ones((16,), jnp.float32)
    full_m = jnp.ones((16,), jnp.bool_)
    tail_m = lax.iota(jnp.int32, 16) < ECH_TAIL
    arrs = [s0, d0, s1, d1, s2, d2]
    for a in range(6):
        base = a * NBP
        pltpu.sync_copy(arrs[a].at[pl.ds(w * ECH, ECH)], ebuf.at[pl.ds(0, ECH)])

        def body(i, base=base):
            idx = ebuf[pl.ds(i * 16, 16)] + base
            plsc.addupdate_scatter(hist, [idx], ones, mask=full_m)

        plsc.parallel_loop(0, ECH_IT, 1, unroll=4)(body)
        idx = ebuf[pl.ds(ECH_IT * 16, 16)] + base
        plsc.addupdate_scatter(hist, [idx], ones, mask=tail_m)
    pltpu.sync_copy(hist, hist_out.at[pl.ds(w * 6 * NBP, 6 * NBP)])


_deg = pl.kernel(
    _deg_body,
    out_type=jax.ShapeDtypeStruct((NW * 6 * NBP,), jnp.float32),
    mesh=_mesh,
    compiler_params=_sc_params,
    scratch_types=[
        pltpu.VMEM((6 * NBP,), jnp.float32),
        pltpu.VMEM((ECH + 16,), jnp.int32),
    ],
)


def _norm_body(part, norms, counts, buf, nbuf, cbuf, dsem):
    w = _wid()
    off = w * NPW
    for a in range(6):
        cps = [pltpu.async_copy(
                   part.at[pl.ds(k * 6 * NBP + a * NBP + off, NPW)],
                   buf.at[pl.ds(k * NPW, NPW)], dsem) for k in range(NW)]
        for cp in cps:
            cp.wait()
        vacc = jnp.zeros((16,), jnp.float32)
        for i in range(NPW // 16):
            s = buf[pl.ds(i * 16, 16)]
            for k in range(1, NW):
                s = s + buf[pl.ds(k * NPW + i * 16, 16)]
            vacc = vacc + s
            x = jnp.maximum(s, 1.0)
            ii = plsc.bitcast(x, jnp.int32)
            yi = 0x5F3759DF - lax.shift_right_logical(ii, 1)
            y = plsc.bitcast(yi, jnp.float32)
            for _ in range(3):
                y = y * (1.5 - 0.5 * x * y * y)
            nbuf[pl.ds(i * 16, 16)] = y
        pltpu.sync_copy(nbuf, norms.at[pl.ds(a * NBP + off, NPW)])
        if a % 2 == 1:
            r = a // 2
            cbuf[pl.ds(0, 16)] = vacc
            pltpu.sync_copy(cbuf, counts.at[pl.ds((r * NW + w) * 16, 16)])


_norm = pl.kernel(
    _norm_body,
    out_type=(jax.ShapeDtypeStruct((6 * NBP,), jnp.float32),
              jax.ShapeDtypeStruct((3 * NW * 16,), jnp.float32)),
    mesh=_mesh,
    compiler_params=_sc_params,
    scratch_types=[
        pltpu.VMEM((NW * NPW,), jnp.float32),
        pltpu.VMEM((NPW,), jnp.float32),
        pltpu.VMEM((16,), jnp.float32),
        pltpu.SemaphoreType.DMA,
    ],
)


def _range_offsets(cbuf, r, w):
    goff = jnp.int32(0)
    ng_w = jnp.int32(0)
    for k in range(NW):
        ck = jnp.sum(cbuf[pl.ds((r * NW + k) * 16, 16)]).astype(jnp.int32)
        ng_k = lax.shift_right_logical(ck + (GB - 1), 7)
        goff = goff + jnp.where(k < w, ng_k, 0)
        ng_w = ng_w + jnp.where(k == w, ng_k, 0)
    return goff, ng_w


def _part_body(norms, counts, s0, d0, s1, d1, s2, d2, w0, w1, w2,
               srcp, dstlp, wfp,
               nflat, cbuf, sblk, dblk, eblk, ps, pd, pw, sem):
    w = _wid()
    lo = w * NPW
    zero16 = jnp.zeros((16,), jnp.float32)
    zi16 = jnp.zeros((16,), jnp.int32)
    pltpu.sync_copy(counts, cbuf)
    srcs = [s0, s1, s2]
    dsts = [d0, d1, d2]
    ws = [w0, w1, w2]
    for r in range(3):
        pltpu.sync_copy(norms.at[pl.ds(2 * r * NBP, NBP)], nflat.at[pl.ds(0, NBP)])
        pltpu.sync_copy(norms.at[pl.ds((2 * r + 1) * NBP, NBP)],
                        nflat.at[pl.ds(NBP, NBP)])
        goff, _ = _range_offsets(cbuf, r, w)
        ebase = r * ECAP + goff * GB

        def blk_body(b, carry, r=r, ebase=ebase):
            rem, flushed = carry
            e0 = b * BLK
            c1 = pltpu.async_copy(srcs[r].at[pl.ds(e0, BLK)], sblk, sem)
            c2 = pltpu.async_copy(dsts[r].at[pl.ds(e0, BLK)], dblk, sem)
            c3 = pltpu.async_copy(ws[r].at[pl.ds(e0, BLK)], eblk, sem)
            c1.wait()
            c2.wait()
            c3.wait()

            def scan(i, base):
                d = dblk[pl.ds(i * 16, 16)]
                m = (d >= lo) & (d < lo + NPW)
                mi = m.astype(jnp.int32)
                pos = base + plsc.cumsum(mi) - mi
                s = sblk[pl.ds(i * 16, 16)]
                ew = eblk[pl.ds(i * 16, 16)]
                ns = plsc.load_gather(nflat, [s])
                nd = plsc.load_gather(nflat, [d + NBP])
                plsc.store_scatter(ps, [pos], s, mask=m)
                plsc.store_scatter(pd, [pos], d - lo, mask=m)
                plsc.store_scatter(pw, [pos], ew * ns * nd, mask=m)
                return base + plsc.all_reduce_population_count(m)

            base = plsc.parallel_loop(
                0, SCAN_IT, 1, unroll=4,
                carry=jnp.zeros((16,), jnp.int32) + rem)(scan)
            cnt = jnp.max(base)
            ngf = lax.shift_right_logical(cnt, 7)

            def flush(g, _, ebase=ebase):
                eout = ebase + (flushed + g) * GB
                c1 = pltpu.async_copy(ps.at[pl.ds(g * GB, GB)],
                                      srcp.at[pl.ds(eout, GB)], sem)
                c2 = pltpu.async_copy(pd.at[pl.ds(g * GB, GB)],
                                      dstlp.at[pl.ds(eout, GB)], sem)
                c3 = pltpu.async_copy(pw.at[pl.ds(g * GB, GB)],
                                      wfp.at[pl.ds(eout, GB)], sem)
                c1.wait()
                c2.wait()
                c3.wait()
                return 0

            lax.fori_loop(0, ngf, flush, 0)
            tail0 = ngf * GB
            for i in range(GB // 16):
                ps[pl.ds(i * 16, 16)] = ps[pl.ds(tail0 + i * 16, 16)]
                pd[pl.ds(i * 16, 16)] = pd[pl.ds(tail0 + i * 16, 16)]
                pw[pl.ds(i * 16, 16)] = pw[pl.ds(tail0 + i * 16, 16)]
            return (cnt & (GB - 1), flushed + ngf)

        rem, flushed = lax.fori_loop(0, NBLK, blk_body,
                                     (jnp.int32(0), jnp.int32(0)))
        for i in range(GB // 16):
            ps[pl.ds(rem + i * 16, 16)] = zi16
            pd[pl.ds(rem + i * 16, 16)] = zi16
            pw[pl.ds(rem + i * 16, 16)] = zero16

        @pl.when(rem > 0)
        def _flush_tail(ebase=ebase, rem=rem, flushed=flushed):
            eout = ebase + flushed * GB
            c1 = pltpu.async_copy(ps.at[pl.ds(0, GB)],
                                  srcp.at[pl.ds(eout, GB)], sem)
            c2 = pltpu.async_copy(pd.at[pl.ds(0, GB)],
                                  dstlp.at[pl.ds(eout, GB)], sem)
            c3 = pltpu.async_copy(pw.at[pl.ds(0, GB)],
                                  wfp.at[pl.ds(eout, GB)], sem)
            c1.wait()
            c2.wait()
            c3.wait()


_part = pl.kernel(
    _part_body,
    out_type=(jax.ShapeDtypeStruct((3 * ECAP,), jnp.int32),
              jax.ShapeDtypeStruct((3 * ECAP,), jnp.int32),
              jax.ShapeDtypeStruct((3 * ECAP,), jnp.float32)),
    mesh=_mesh,
    compiler_params=_sc_params,
    scratch_types=[
        pltpu.VMEM((2 * NBP,), jnp.float32),
        pltpu.VMEM((3 * NW * 16,), jnp.float32),
        pltpu.VMEM((BLK,), jnp.int32),
        pltpu.VMEM((BLK,), jnp.int32),
        pltpu.VMEM((BLK,), jnp.float32),
        pltpu.VMEM((PEND,), jnp.int32),
        pltpu.VMEM((PEND,), jnp.int32),
        pltpu.VMEM((PEND,), jnp.float32),
        pltpu.SemaphoreType.DMA,
    ],
)


def _agg_body(do_relu, Z0, Z1, Z2, srcp, dstlp, wfp, counts, b0, b1, b2,
              out, acc, cbuf, sl, dl, wl, stage, bbuf, sem, sem_a, sem_b):
    w = _wid()
    lo = w * NPW
    zero16 = jnp.zeros((16,), jnp.float32)
    iota16 = lax.iota(jnp.int32, 16)

    def za(i):
        acc[pl.ds(i * 16, 16)] = zero16

    plsc.parallel_loop(0, (NPW * D) // 16, 1, unroll=8)(za)
    pltpu.sync_copy(counts, cbuf)
    Zs = [Z0, Z1, Z2]
    for r in range(3):
        goff, ng_w = _range_offsets(cbuf, r, w)
        ebase = r * ECAP + goff * GB
        nch = lax.shift_right_logical(ng_w * GB + (CH - 1), 10)

        def ch_body(c, _, r=r, ebase=ebase, ng_w=ng_w):
            c0 = ebase + c * CH
            d1_ = pltpu.async_copy(srcp.at[pl.ds(c0, CH)], sl, sem)
            d2_ = pltpu.async_copy(dstlp.at[pl.ds(c0, CH)], dl, sem)
            d3_ = pltpu.async_copy(wfp.at[pl.ds(c0, CH)], wl, sem)
            d1_.wait()
            d2_.wait()
            d3_.wait()
            ngc = jnp.minimum(ng_w - c * (CH // GB), CH // GB)
            nh = ngc * 2

            def fire(h, slot, r=r):
                sm = sem_a if slot == 0 else sem_b
                pltpu.async_copy(
                    Zs[r].at[sl.at[pl.ds(h * HG, HG)]],
                    stage.at[pl.ds(slot * HG, HG), :], sm)

            def drain(slot, r=r):
                sm = sem_a if slot == 0 else sem_b
                pltpu.make_async_copy(
                    Zs[r].at[pl.ds(0, HG), :],
                    stage.at[pl.ds(slot * HG, HG), :], sm).wait()

            def accum(h, slot):
                def ed(j):
                    js = jnp.full((16,), h * HG + j, jnp.int32)
                    wb = plsc.load_gather(wl, [js])
                    db = plsc.load_gather(dl, [js]) * D
                    rowv = jnp.full((16,), slot * HG + j, jnp.int32)
                    colv = iota16
                    for k in range(16):
                        v = plsc.load_gather(stage, [rowv, colv])
                        plsc.addupdate_scatter(acc, [db + colv], v * wb)
                        colv = colv + 16

                plsc.parallel_loop(0, HG, 1, unroll=4)(ed)

            fire(0, 0)

            def pair(g, _):
                h0 = g * 2
                drain(0)
                fire(h0 + 1, 1)
                accum(h0, 0)
                drain(1)

                @pl.when(g + 1 < ngc)
                def _():
                    fire(h0 + 2, 0)

                accum(h0 + 1, 1)
                return 0

            lax.fori_loop(0, ngc, pair, 0)
            return 0

        lax.fori_loop(0, nch, ch_body, 0)

    pltpu.sync_copy(b0, bbuf.at[pl.ds(0, D)])
    pltpu.sync_copy(b1, bbuf.at[pl.ds(D, D)])
    pltpu.sync_copy(b2, bbuf.at[pl.ds(2 * D, D)])
    for k in range(16):
        bbuf[pl.ds(3 * D + k * 16, 16)] = (
            bbuf[pl.ds(k * 16, 16)] + bbuf[pl.ds(D + k * 16, 16)]
            + bbuf[pl.ds(2 * D + k * 16, 16)])

    def ep(t, _):
        rbase = t * D
        for k in range(16):
            v = acc[pl.ds(rbase + k * 16, 16)] + bbuf[pl.ds(3 * D + k * 16, 16)]
            if do_relu:
                v = jnp.maximum(v, 0.0)
            acc[pl.ds(rbase + k * 16, 16)] = v
        return 0

    lax.fori_loop(0, NPW, ep, 0)
    pltpu.sync_copy(acc, out.at[pl.ds(lo * D, NPW * D)])


def _make_agg(do_relu):
    return pl.kernel(
        functools.partial(_agg_body, do_relu),
        out_type=jax.ShapeDtypeStruct((NBP * D,), jnp.float32),
        mesh=_mesh,
        compiler_params=_sc_params,
        scratch_types=[
            pltpu.VMEM((NPW * D,), jnp.float32),
            pltpu.VMEM((3 * NW * 16,), jnp.float32),
            pltpu.VMEM((CH,), jnp.int32),
            pltpu.VMEM((CH,), jnp.int32),
            pltpu.VMEM((CH,), jnp.float32),
            pltpu.VMEM((GB, D), jnp.float32),
            pltpu.VMEM((4 * D,), jnp.float32),
            pltpu.SemaphoreType.DMA,
            pltpu.SemaphoreType.DMA,
            pltpu.SemaphoreType.DMA,
        ],
    )


_agg_relu = _make_agg(True)
_agg_plain = _make_agg(False)


def _mm_body(x_ref, w0_ref, w1_ref, w2_ref, o0_ref, o1_ref, o2_ref):
    xb = x_ref[...]
    o0_ref[...] = jnp.dot(xb, w0_ref[...], preferred_element_type=jnp.float32)
    o1_ref[...] = jnp.dot(xb, w1_ref[...], preferred_element_type=jnp.float32)
    o2_ref[...] = jnp.dot(xb, w2_ref[...], preferred_element_type=jnp.float32)


def _make_mm(m, bm):
    wspec = pl.BlockSpec((D, D), lambda i: (0, 0))
    ospec = pl.BlockSpec((bm, D), lambda i: (i, 0))
    return pl.pallas_call(
        _mm_body,
        grid=(m // bm,),
        in_specs=[pl.BlockSpec((bm, D), lambda i: (i, 0)), wspec, wspec, wspec],
        out_specs=[ospec, ospec, ospec],
        out_shape=[jax.ShapeDtypeStruct((m, D), jnp.float32)] * 3,
    )


_mm_a = _make_mm(N, MMB_A)
_mm_b = _make_mm(NBP, MMB_B)


def kernel(x, edge_index_r0, edge_index_r1, edge_index_r2, w_r0, w_r1, w_r2,
           W1_r0, b1_r0, W1_r1, b1_r1, W1_r2, b1_r2,
           W2_r0, b2_r0, W2_r1, b2_r1, W2_r2, b2_r2):
    sd = (edge_index_r0[0], edge_index_r0[1], edge_index_r1[0],
          edge_index_r1[1], edge_index_r2[0], edge_index_r2[1])
    part = _deg(*sd)
    norms, counts = _norm(part)
    srcp, dstlp, wfp = _part(norms, counts, *sd, w_r0, w_r1, w_r2)
    z0, z1, z2 = _mm_a(x, W1_r0, W1_r1, W1_r2)
    hflat = _agg_relu(z0, z1, z2, srcp, dstlp, wfp, counts,
                      b1_r0, b1_r1, b1_r2)
    h = hflat.reshape(NBP, D)
    y0, y1, y2 = _mm_b(h, W2_r0, W2_r1, W2_r2)
    oflat = _agg_plain(y0, y1, y2, srcp, dstlp, wfp, counts,
                       b2_r0, b2_r1, b2_r2)
    return oflat.reshape(NBP, D)[:N]
